# baseline (speedup 1.0000x reference)
import dataclasses
import functools

import jax
import jax.numpy as jnp
from jax import lax
from jax.experimental import pallas as pl
from jax.experimental.pallas import tpu as pltpu
from jax.experimental.pallas import tpu_sc as plsc

N = 10000
E = 320000
D = 128
NC = 2
NS = 16
NW = NC * NS
CHUNK = 64
CH0 = 272
CH1 = 48
IB = 8
NB0 = CH0 // IB
NB1 = CH1 // IB
E_PAD = NS * (CH0 + CH1) * CHUNK
EPT = E_PAD // NW
AR = 10240
ROWS_PER_TILE = AR // NS
DR = AR // 128
EPS = 1e-3

_mesh = plsc.VectorSubcoreMesh(core_axis_name="c", subcore_axis_name="s")

_sc_params = pltpu.CompilerParams()
if "needs_layout_passes" in pltpu.CompilerParams.__dataclass_fields__:
    _sc_params = dataclasses.replace(_sc_params, needs_layout_passes=False)


@functools.partial(
    pl.kernel,
    mesh=_mesh,
    out_type=jax.ShapeDtypeStruct((NC, DR, 128), jnp.float32),
    scratch_types=[
        pltpu.VMEM((EPT // 5,), jnp.int32),
        pltpu.VMEM((DR, 128), jnp.float32),
        pltpu.VMEM((1, DR), jnp.int32),
        pltpu.VMEM_SHARED((DR, 128), jnp.float32),
    ],
    compiler_params=_sc_params,
)
def _deg_kernel(row_hbm, out_hbm, idx_v, hist_v, iota_v, shared_deg):
    c = lax.axis_index("c")
    s = lax.axis_index("s")
    wid = s * NC + c
    base = wid * EPT

    @pl.loop(0, DR)
    def _zero(i):
        @pl.loop(0, 8)
        def _zl(q):
            off = pl.multiple_of(q * 16, 16)
            hist_v[i, pl.ds(off, 16)] = jnp.zeros((16,), jnp.float32)

    @pl.when(s == 0)
    def _():
        pltpu.sync_copy(hist_v, shared_deg)

    base16 = lax.iota(jnp.int32, 16)

    @pl.loop(0, DR // 16)
    def _iota(j):
        off = pl.multiple_of(j * 16, 16)
        iota_v[0, pl.ds(off, 16)] = base16 + j * 16

    plsc.subcore_barrier()

    ones = jnp.full((16,), 1.0, jnp.float32)

    @pl.loop(0, 5)
    def _outer(t):
        hoff = pl.multiple_of(base + t * (EPT // 5), 8)
        pltpu.sync_copy(row_hbm.at[pl.ds(hoff, EPT // 5)], idx_v)

        @pl.loop(0, EPT // 5 // 16)
        def _acc(k):
            off = pl.multiple_of(k * 16, 16)
            idx16 = idx_v[pl.ds(off, 16)]
            r = lax.shift_right_logical(idx16, 7)
            cc = lax.bitwise_and(idx16, 127)
            plsc.addupdate_scatter(hist_v, [r, cc], ones)

    pltpu.sync_copy(hist_v, shared_deg.at[iota_v.at[0]], add=True)

    plsc.subcore_barrier()

    @pl.when(s == 0)
    def _():
        pltpu.sync_copy(shared_deg, out_hbm.at[c])


@functools.partial(
    pl.kernel,
    mesh=_mesh,
    out_type=jax.ShapeDtypeStruct((NC, AR, D), jnp.float32),
    scratch_types=[
        pltpu.VMEM((IB, CHUNK), jnp.int32),
        pltpu.VMEM((IB, CHUNK), jnp.int32),
        pltpu.VMEM((IB, CHUNK), jnp.int32),
        pltpu.VMEM((IB, CHUNK), jnp.int32),
        pltpu.VMEM((CHUNK, D), jnp.float32),
        pltpu.VMEM((CHUNK, D), jnp.float32),
        pltpu.VMEM((CHUNK, D), jnp.float32),
        pltpu.VMEM((CHUNK, D), jnp.float32),
        pltpu.VMEM_SHARED((AR, D), jnp.float32),
        pltpu.SemaphoreType.DMA,
        pltpu.SemaphoreType.DMA,
        pltpu.SemaphoreType.DMA,
        pltpu.SemaphoreType.DMA,
        pltpu.SemaphoreType.DMA,
        pltpu.SemaphoreType.DMA,
    ],
    compiler_params=_sc_params,
)
def _agg_kernel(hs_hbm, rowi_hbm, coli_hbm, out_hbm,
                ibr0, ibc0, ibr1, ibc1,
                rows0_v, rows1_v, rows2_v, rows3_v, shared_a,
                sem0, sem1, sem2, sem3, si0, si1):
    c = lax.axis_index("c")
    s = lax.axis_index("s")
    base = pl.multiple_of(jnp.where(c == 0, s * CH0, NS * CH0 + s * CH1), 8)
    nb = jnp.where(c == 0, NB0, NB1)

    pltpu.async_copy(rowi_hbm.at[pl.ds(base, IB)], ibr0, si0)
    pltpu.async_copy(coli_hbm.at[pl.ds(base, IB)], ibc0, si0)
    off1 = pl.multiple_of(base + IB, 8)
    pltpu.async_copy(rowi_hbm.at[pl.ds(off1, IB)], ibr1, si1)
    pltpu.async_copy(coli_hbm.at[pl.ds(off1, IB)], ibc1, si1)

    @pl.loop(0, CHUNK)
    def _z0(i):
        @pl.loop(0, D // 16)
        def _z1(q):
            off = pl.multiple_of(q * 16, 16)
            rows0_v[i, pl.ds(off, 16)] = jnp.zeros((16,), jnp.float32)

    @pl.loop(0, ROWS_PER_TILE // CHUNK)
    def _zs(k):
        off = pl.multiple_of(s * ROWS_PER_TILE + k * CHUNK, 8)
        pltpu.sync_copy(rows0_v, shared_a.at[pl.ds(off, CHUNK)])

    plsc.subcore_barrier()

    def wait_idx(br, bc, sem):
        pltpu.make_async_copy(rowi_hbm.at[pl.ds(base, IB)], br, sem).wait()
        pltpu.make_async_copy(coli_hbm.at[pl.ds(base, IB)], bc, sem).wait()

    bufs = (rows0_v, rows1_v, rows2_v, rows3_v)
    sems = (sem0, sem1, sem2, sem3)

    def process_block(br, bc):
        for q in range(4):
            pltpu.async_copy(hs_hbm.at[br.at[q]], bufs[q], sems[q])
        for q in range(4):
            pltpu.make_async_copy(hs_hbm.at[br.at[q]], bufs[q],
                                  sems[q]).wait()
            pltpu.sync_copy(bufs[q], shared_a.at[bc.at[q]], add=True)
            pltpu.async_copy(hs_hbm.at[br.at[q + 4]], bufs[q], sems[q])
        for q in range(4):
            pltpu.make_async_copy(hs_hbm.at[br.at[q + 4]], bufs[q],
                                  sems[q]).wait()
            pltpu.sync_copy(bufs[q], shared_a.at[bc.at[q + 4]], add=True)

    @pl.loop(0, nb // 2)
    def _blocks(t):
        b = t * 2
        wait_idx(ibr0, ibc0, si0)
        process_block(ibr0, ibc0)

        @pl.when(b + 2 < nb)
        def _():
            off = pl.multiple_of(base + (b + 2) * IB, 8)
            pltpu.async_copy(rowi_hbm.at[pl.ds(off, IB)], ibr0, si0)
            pltpu.async_copy(coli_hbm.at[pl.ds(off, IB)], ibc0, si0)

        wait_idx(ibr1, ibc1, si1)
        process_block(ibr1, ibc1)

        @pl.when(b + 3 < nb)
        def _():
            off = pl.multiple_of(base + (b + 3) * IB, 8)
            pltpu.async_copy(rowi_hbm.at[pl.ds(off, IB)], ibr1, si1)
            pltpu.async_copy(coli_hbm.at[pl.ds(off, IB)], ibc1, si1)

    plsc.subcore_barrier()

    off = pl.multiple_of(s * ROWS_PER_TILE, 8)
    pltpu.sync_copy(
        shared_a.at[pl.ds(off, ROWS_PER_TILE)],
        out_hbm.at[c].at[pl.ds(off, ROWS_PER_TILE)],
    )


def _tc1_body(deg_ref, x_ref, wg_ref, ws_ref, gamma_ref, var_ref,
              hs_ref, dd_ref):
    sc = gamma_ref[...] * lax.rsqrt(var_ref[...] + EPS)
    wg = wg_ref[...] * sc
    ws = ws_ref[...] * sc
    deg = deg_ref[..., 0:1] + deg_ref[..., 1:2]
    dinv = jnp.where(deg > 0.0, lax.rsqrt(deg), 0.0)
    xb = x_ref[...]
    hp = jnp.dot(xb, wg, preferred_element_type=jnp.float32,
                 precision=lax.Precision.HIGHEST)
    hs_ref[...] = hp * dinv
    dd_ref[...] = hp + jnp.dot(xb, ws, preferred_element_type=jnp.float32,
                               precision=lax.Precision.HIGHEST)


def _tc2_body(a_ref, dd_ref, deg_ref, gamma_ref, var_ref, beta_ref,
              bias_ref, mm_ref, out_ref):
    sc = gamma_ref[...] * lax.rsqrt(var_ref[...] + EPS)
    t = beta_ref[...] + sc * (bias_ref[...] - mm_ref[...])
    deg = deg_ref[..., 0:1] + deg_ref[..., 1:2]
    dinv = jnp.where(deg > 0.0, lax.rsqrt(deg), 0.0)
    a = a_ref[0] + a_ref[1]
    out_ref[...] = dinv * a + dd_ref[...] + t


RB = 400


def kernel(x, edge_index, W_gcn, W_self, bias, gamma, beta, moving_mean,
           moving_var):
    row = edge_index[0]
    col = edge_index[1]

    pad = E_PAD - E
    row_deg = jnp.concatenate([row, jnp.full((pad,), AR - 1, jnp.int32)])
    row_g = jnp.concatenate([row, jnp.zeros((pad,), jnp.int32)])
    col_s = jnp.concatenate([col, jnp.full((pad,), N, jnp.int32)])
    row_g2 = row_g.reshape(NS * (CH0 + CH1), CHUNK)
    col_s2 = col_s.reshape(NS * (CH0 + CH1), CHUNK)

    deg2 = _deg_kernel(row_deg)
    degT = deg2.reshape(NC, AR)[:, :N].T

    g2 = gamma.reshape(1, D)
    v2 = moving_var.reshape(1, D)
    b2 = beta.reshape(1, D)
    bi2 = bias.reshape(1, D)
    m2 = moving_mean.reshape(1, D)

    grid = (N // RB,)
    vec_spec = pl.BlockSpec((1, D), lambda i: (0, 0))
    hs, dd = pl.pallas_call(
        _tc1_body,
        grid=grid,
        in_specs=[
            pl.BlockSpec((RB, 2), lambda i: (i, 0)),
            pl.BlockSpec((RB, D), lambda i: (i, 0)),
            pl.BlockSpec((D, D), lambda i: (0, 0)),
            pl.BlockSpec((D, D), lambda i: (0, 0)),
            vec_spec,
            vec_spec,
        ],
        out_specs=[
            pl.BlockSpec((RB, D), lambda i: (i, 0)),
            pl.BlockSpec((RB, D), lambda i: (i, 0)),
        ],
        out_shape=[
            jax.ShapeDtypeStruct((N, D), jnp.float32),
            jax.ShapeDtypeStruct((N, D), jnp.float32),
        ],
    )(degT, x, W_gcn, W_self, g2, v2)

    a2 = _agg_kernel(hs, row_g2, col_s2)

    out = pl.pallas_call(
        _tc2_body,
        grid=grid,
        in_specs=[
            pl.BlockSpec((NC, RB, D), lambda i: (0, i, 0)),
            pl.BlockSpec((RB, D), lambda i: (i, 0)),
            pl.BlockSpec((RB, 2), lambda i: (i, 0)),
            vec_spec, vec_spec, vec_spec, vec_spec, vec_spec,
        ],
        out_specs=pl.BlockSpec((RB, D), lambda i: (i, 0)),
        out_shape=jax.ShapeDtypeStruct((N, D), jnp.float32),
    )(a2, dd, degT, g2, v2, b2, bi2, m2)

    return out


# speedup vs baseline: 1.1596x; 1.1596x over previous
import dataclasses
import functools

import jax
import jax.numpy as jnp
from jax import lax
from jax.experimental import pallas as pl
from jax.experimental.pallas import tpu as pltpu
from jax.experimental.pallas import tpu_sc as plsc

N = 10000
E = 320000
D = 128
NC = 2
NS = 16
NW = NC * NS
CHUNK = 64
CH0 = 304
CH1 = 16
IB = 8
NB0 = CH0 // IB
NB1 = CH1 // IB
E_PAD = NS * (CH0 + CH1) * CHUNK
EPT = E_PAD // NW
AR = 10240
ROWS_PER_TILE = AR // NS
DR = AR // 128
EPS = 1e-3

_mesh = plsc.VectorSubcoreMesh(core_axis_name="c", subcore_axis_name="s")

_sc_params = pltpu.CompilerParams()
if "needs_layout_passes" in pltpu.CompilerParams.__dataclass_fields__:
    _sc_params = dataclasses.replace(_sc_params, needs_layout_passes=False)


@functools.partial(
    pl.kernel,
    mesh=_mesh,
    out_type=jax.ShapeDtypeStruct((NC, DR, 128), jnp.float32),
    scratch_types=[
        pltpu.VMEM((EPT // 5,), jnp.int32),
        pltpu.VMEM((DR, 128), jnp.float32),
        pltpu.VMEM((1, DR), jnp.int32),
        pltpu.VMEM_SHARED((DR, 128), jnp.float32),
    ],
    compiler_params=_sc_params,
)
def _deg_kernel(row_hbm, out_hbm, idx_v, hist_v, iota_v, shared_deg):
    c = lax.axis_index("c")
    s = lax.axis_index("s")
    wid = s * NC + c
    base = wid * EPT

    @pl.loop(0, DR)
    def _zero(i):
        @pl.loop(0, 8)
        def _zl(q):
            off = pl.multiple_of(q * 16, 16)
            hist_v[i, pl.ds(off, 16)] = jnp.zeros((16,), jnp.float32)

    @pl.when(s == 0)
    def _():
        pltpu.sync_copy(hist_v, shared_deg)

    base16 = lax.iota(jnp.int32, 16)

    @pl.loop(0, DR // 16)
    def _iota(j):
        off = pl.multiple_of(j * 16, 16)
        iota_v[0, pl.ds(off, 16)] = base16 + j * 16

    plsc.subcore_barrier()

    ones = jnp.full((16,), 1.0, jnp.float32)

    @pl.loop(0, 5)
    def _outer(t):
        hoff = pl.multiple_of(base + t * (EPT // 5), 8)
        pltpu.sync_copy(row_hbm.at[pl.ds(hoff, EPT // 5)], idx_v)

        @pl.loop(0, EPT // 5 // 16)
        def _acc(k):
            off = pl.multiple_of(k * 16, 16)
            idx16 = idx_v[pl.ds(off, 16)]
            r = lax.shift_right_logical(idx16, 7)
            cc = lax.bitwise_and(idx16, 127)
            plsc.addupdate_scatter(hist_v, [r, cc], ones)

    pltpu.sync_copy(hist_v, shared_deg.at[iota_v.at[0]], add=True)

    plsc.subcore_barrier()

    @pl.when(s == 0)
    def _():
        pltpu.sync_copy(shared_deg, out_hbm.at[c])


@functools.partial(
    pl.kernel,
    mesh=_mesh,
    out_type=jax.ShapeDtypeStruct((NC, AR, D), jnp.float32),
    scratch_types=[
        pltpu.VMEM((IB, CHUNK), jnp.int32),
        pltpu.VMEM((IB, CHUNK), jnp.int32),
        pltpu.VMEM((IB, CHUNK), jnp.int32),
        pltpu.VMEM((IB, CHUNK), jnp.int32),
        pltpu.VMEM((CHUNK, D), jnp.float32),
        pltpu.VMEM((CHUNK, D), jnp.float32),
        pltpu.VMEM((CHUNK, D), jnp.float32),
        pltpu.VMEM((CHUNK, D), jnp.float32),
        pltpu.VMEM_SHARED((AR, D), jnp.float32),
        pltpu.SemaphoreType.DMA,
        pltpu.SemaphoreType.DMA,
        pltpu.SemaphoreType.DMA,
        pltpu.SemaphoreType.DMA,
        pltpu.SemaphoreType.DMA,
        pltpu.SemaphoreType.DMA,
    ],
    compiler_params=_sc_params,
)
def _agg_kernel(hs_hbm, rowi_hbm, coli_hbm, out_hbm,
                ibr0, ibc0, ibr1, ibc1,
                rows0_v, rows1_v, rows2_v, rows3_v, shared_a,
                sem0, sem1, sem2, sem3, si0, si1):
    c = lax.axis_index("c")
    s = lax.axis_index("s")
    base = pl.multiple_of(jnp.where(c == 0, s * CH0, NS * CH0 + s * CH1), 8)
    nb = jnp.where(c == 0, NB0, NB1)

    pltpu.async_copy(rowi_hbm.at[pl.ds(base, IB)], ibr0, si0)
    pltpu.async_copy(coli_hbm.at[pl.ds(base, IB)], ibc0, si0)
    off1 = pl.multiple_of(base + IB, 8)
    pltpu.async_copy(rowi_hbm.at[pl.ds(off1, IB)], ibr1, si1)
    pltpu.async_copy(coli_hbm.at[pl.ds(off1, IB)], ibc1, si1)

    @pl.loop(0, CHUNK)
    def _z0(i):
        @pl.loop(0, D // 16)
        def _z1(q):
            off = pl.multiple_of(q * 16, 16)
            rows0_v[i, pl.ds(off, 16)] = jnp.zeros((16,), jnp.float32)

    @pl.loop(0, ROWS_PER_TILE // CHUNK)
    def _zs(k):
        off = pl.multiple_of(s * ROWS_PER_TILE + k * CHUNK, 8)
        pltpu.sync_copy(rows0_v, shared_a.at[pl.ds(off, CHUNK)])

    plsc.subcore_barrier()

    def wait_idx(br, bc, sem):
        pltpu.make_async_copy(rowi_hbm.at[pl.ds(base, IB)], br, sem).wait()
        pltpu.make_async_copy(coli_hbm.at[pl.ds(base, IB)], bc, sem).wait()

    bufs = (rows0_v, rows1_v, rows2_v, rows3_v)
    sems = (sem0, sem1, sem2, sem3)

    def process_block(br, bc):
        for q in range(4):
            pltpu.async_copy(hs_hbm.at[br.at[q]], bufs[q], sems[q])
        for q in range(4):
            pltpu.make_async_copy(hs_hbm.at[br.at[q]], bufs[q],
                                  sems[q]).wait()
            pltpu.sync_copy(bufs[q], shared_a.at[bc.at[q]], add=True)
            pltpu.async_copy(hs_hbm.at[br.at[q + 4]], bufs[q], sems[q])
        for q in range(4):
            pltpu.make_async_copy(hs_hbm.at[br.at[q + 4]], bufs[q],
                                  sems[q]).wait()
            pltpu.sync_copy(bufs[q], shared_a.at[bc.at[q + 4]], add=True)

    @pl.loop(0, nb // 2)
    def _blocks(t):
        b = t * 2
        wait_idx(ibr0, ibc0, si0)
        process_block(ibr0, ibc0)

        @pl.when(b + 2 < nb)
        def _():
            off = pl.multiple_of(base + (b + 2) * IB, 8)
            pltpu.async_copy(rowi_hbm.at[pl.ds(off, IB)], ibr0, si0)
            pltpu.async_copy(coli_hbm.at[pl.ds(off, IB)], ibc0, si0)

        wait_idx(ibr1, ibc1, si1)
        process_block(ibr1, ibc1)

        @pl.when(b + 3 < nb)
        def _():
            off = pl.multiple_of(base + (b + 3) * IB, 8)
            pltpu.async_copy(rowi_hbm.at[pl.ds(off, IB)], ibr1, si1)
            pltpu.async_copy(coli_hbm.at[pl.ds(off, IB)], ibc1, si1)

    plsc.subcore_barrier()

    off = pl.multiple_of(s * ROWS_PER_TILE, 8)
    pltpu.sync_copy(
        shared_a.at[pl.ds(off, ROWS_PER_TILE)],
        out_hbm.at[c].at[pl.ds(off, ROWS_PER_TILE)],
    )


def _tc1_body(deg_ref, x_ref, wg_ref, ws_ref, gamma_ref, var_ref,
              hs_ref, dd_ref):
    sc = gamma_ref[...] * lax.rsqrt(var_ref[...] + EPS)
    wg = wg_ref[...] * sc
    ws = ws_ref[...] * sc
    deg = deg_ref[..., 0:1] + deg_ref[..., 1:2]
    dinv = jnp.where(deg > 0.0, lax.rsqrt(deg), 0.0)
    xb = x_ref[...]
    hp = jnp.dot(xb, wg, preferred_element_type=jnp.float32,
                 precision=lax.Precision.HIGHEST)
    hs_ref[...] = hp * dinv
    dd_ref[...] = hp + jnp.dot(xb, ws, preferred_element_type=jnp.float32,
                               precision=lax.Precision.HIGHEST)


def _tc2_body(a_ref, dd_ref, deg_ref, gamma_ref, var_ref, beta_ref,
              bias_ref, mm_ref, out_ref):
    sc = gamma_ref[...] * lax.rsqrt(var_ref[...] + EPS)
    t = beta_ref[...] + sc * (bias_ref[...] - mm_ref[...])
    deg = deg_ref[..., 0:1] + deg_ref[..., 1:2]
    dinv = jnp.where(deg > 0.0, lax.rsqrt(deg), 0.0)
    a = a_ref[0] + a_ref[1]
    out_ref[...] = dinv * a + dd_ref[...] + t


RB = 400


def kernel(x, edge_index, W_gcn, W_self, bias, gamma, beta, moving_mean,
           moving_var):
    row = edge_index[0]
    col = edge_index[1]

    pad = E_PAD - E
    row_deg = jnp.concatenate([row, jnp.full((pad,), AR - 1, jnp.int32)])
    row_g = jnp.concatenate([row, jnp.zeros((pad,), jnp.int32)])
    col_s = jnp.concatenate([col, jnp.full((pad,), N, jnp.int32)])
    row_g2 = row_g.reshape(NS * (CH0 + CH1), CHUNK)
    col_s2 = col_s.reshape(NS * (CH0 + CH1), CHUNK)

    deg2 = _deg_kernel(row_deg)
    degT = deg2.reshape(NC, AR)[:, :N].T

    g2 = gamma.reshape(1, D)
    v2 = moving_var.reshape(1, D)
    b2 = beta.reshape(1, D)
    bi2 = bias.reshape(1, D)
    m2 = moving_mean.reshape(1, D)

    grid = (N // RB,)
    vec_spec = pl.BlockSpec((1, D), lambda i: (0, 0))
    hs, dd = pl.pallas_call(
        _tc1_body,
        grid=grid,
        in_specs=[
            pl.BlockSpec((RB, 2), lambda i: (i, 0)),
            pl.BlockSpec((RB, D), lambda i: (i, 0)),
            pl.BlockSpec((D, D), lambda i: (0, 0)),
            pl.BlockSpec((D, D), lambda i: (0, 0)),
            vec_spec,
            vec_spec,
        ],
        out_specs=[
            pl.BlockSpec((RB, D), lambda i: (i, 0)),
            pl.BlockSpec((RB, D), lambda i: (i, 0)),
        ],
        out_shape=[
            jax.ShapeDtypeStruct((N, D), jnp.float32),
            jax.ShapeDtypeStruct((N, D), jnp.float32),
        ],
    )(degT, x, W_gcn, W_self, g2, v2)

    a2 = _agg_kernel(hs, row_g2, col_s2)

    out = pl.pallas_call(
        _tc2_body,
        grid=grid,
        in_specs=[
            pl.BlockSpec((NC, RB, D), lambda i: (0, i, 0)),
            pl.BlockSpec((RB, D), lambda i: (i, 0)),
            pl.BlockSpec((RB, 2), lambda i: (i, 0)),
            vec_spec, vec_spec, vec_spec, vec_spec, vec_spec,
        ],
        out_specs=pl.BlockSpec((RB, D), lambda i: (i, 0)),
        out_shape=jax.ShapeDtypeStruct((N, D), jnp.float32),
    )(a2, dd, degT, g2, v2, b2, bi2, m2)

    return out
